# baseline (speedup 1.0000x reference)
import functools

import jax
import jax.numpy as jnp
from jax import lax
from jax.experimental import pallas as pl
from jax.experimental.pallas import tpu as pltpu
from jax.experimental.pallas import tpu_sc as plsc

T = 2048
D = 2048
E = 8
R = 32
SCALING = 64 / 32
ER = E * R

TILE_T = 256

_INFO = plsc.get_sparse_core_info()
_NC, _NS, _L = _INFO.num_cores, _INFO.num_subcores, _INFO.num_lanes
_NW = _NC * _NS
_TPW = T // _NW


def _logits_kernel(x_ref, wg_ref, l_ref):
    l_ref[...] = jnp.dot(x_ref[...], wg_ref[...],
                         preferred_element_type=jnp.float32)


def _lane_perm(v, idx):
    return lax.gather(
        v, idx[:, None],
        lax.GatherDimensionNumbers(
            offset_dims=(), collapsed_slice_dims=(0,), start_index_map=(0,)),
        (1,), mode=lax.GatherScatterMode.PROMISE_IN_BOUNDS)


def _sc_router_kernel(l_hbm, w_hbm, lv, wv, sem):
    wid = lax.axis_index("s") * _NC + lax.axis_index("c")
    base = wid * _TPW * E
    pltpu.sync_copy(l_hbm.at[pl.ds(base, _TPW * E)], lv)
    lanes = jnp.arange(_L, dtype=jnp.int32)
    lane8 = lanes & 7
    perms = [lanes ^ k for k in (1, 2, 4)]
    neg = jnp.full((_L,), -jnp.inf, jnp.float32)

    def group_max(v):
        for p in perms:
            v = jnp.maximum(v, _lane_perm(v, p))
        return v

    def group_min(v):
        for p in perms:
            v = jnp.minimum(v, _lane_perm(v, p))
        return v

    for j in range(_TPW * E // _L):
        v = lv[pl.ds(j * _L, _L)]
        m1 = group_max(v)
        i1 = group_min(jnp.where(v == m1, lane8, E))
        vm = jnp.where(lane8 == i1, neg, v)
        m2 = group_max(vm)
        i2 = group_min(jnp.where(vm == m2, lane8, E))
        e2 = jnp.exp(m2 - m1)
        denom = 1.0 + e2
        w = (jnp.where(lane8 == i1, 1.0, 0.0)
             + jnp.where(lane8 == i2, e2, 0.0)) / denom
        wv[pl.ds(j * _L, _L)] = w
    pltpu.sync_copy(wv, w_hbm.at[pl.ds(base, _TPW * E)])


def _fused_kernel(x_ref, wb_ref, w_ref, a2_ref, b2_ref, o_ref):
    x = x_ref[...]
    w = w_ref[...]
    erow = jax.lax.broadcasted_iota(jnp.int32, (E, ER), 0)
    ecol = jax.lax.broadcasted_iota(jnp.int32, (E, ER), 1) // R
    expand = (erow == ecol).astype(jnp.float32)
    w_cols = jnp.dot(w, expand, preferred_element_type=jnp.float32)

    a = jnp.dot(x, a2_ref[...], preferred_element_type=jnp.float32)
    moe = jnp.dot(a * w_cols, b2_ref[...], preferred_element_type=jnp.float32)
    base = jnp.dot(x, wb_ref[...], preferred_element_type=jnp.float32)
    o_ref[...] = base + moe * SCALING


@jax.jit
def kernel(hidden_states, W_base, W_gate, lora_A, lora_B):
    A_all = lora_A.reshape(ER, D).T
    B_all = lora_B.transpose(0, 2, 1).reshape(ER, D)

    logits = pl.pallas_call(
        _logits_kernel,
        grid=(T // TILE_T,),
        in_specs=[
            pl.BlockSpec((TILE_T, D), lambda i: (i, 0)),
            pl.BlockSpec((D, E), lambda i: (0, 0)),
        ],
        out_specs=pl.BlockSpec((TILE_T, E), lambda i: (i, 0)),
        out_shape=jax.ShapeDtypeStruct((T, E), jnp.float32),
    )(hidden_states, W_gate)

    sc_router = functools.partial(
        pl.kernel,
        out_type=jax.ShapeDtypeStruct((T * E,), jnp.float32),
        mesh=plsc.VectorSubcoreMesh(core_axis_name="c", subcore_axis_name="s"),
        scratch_types=[
            pltpu.VMEM((_TPW * E,), jnp.float32),
            pltpu.VMEM((_TPW * E,), jnp.float32),
            pltpu.SemaphoreType.DMA,
        ],
    )(_sc_router_kernel)
    routing_w = sc_router(logits.reshape(T * E)).reshape(T, E)

    return pl.pallas_call(
        _fused_kernel,
        grid=(T // TILE_T,),
        in_specs=[
            pl.BlockSpec((TILE_T, D), lambda i: (i, 0)),
            pl.BlockSpec((D, D), lambda i: (0, 0)),
            pl.BlockSpec((TILE_T, E), lambda i: (i, 0)),
            pl.BlockSpec((D, ER), lambda i: (0, 0)),
            pl.BlockSpec((ER, D), lambda i: (0, 0)),
        ],
        out_specs=pl.BlockSpec((TILE_T, D), lambda i: (i, 0)),
        out_shape=jax.ShapeDtypeStruct((T, D), jnp.float32),
    )(hidden_states, W_base, routing_w, A_all, B_all)


# speedup vs baseline: 1.7112x; 1.7112x over previous
import jax
import jax.numpy as jnp
from jax.experimental import pallas as pl

T = 2048
D = 2048
E = 8
R = 32
SCALING = 64 / 32
ER = E * R

TILE_T = 512


def _fused_kernel(x_ref, wb_ref, wg_ref, a2_ref, b2_ref, o_ref):
    x = x_ref[...]
    logits = jnp.dot(x, wg_ref[...], preferred_element_type=jnp.float32)
    cols = jax.lax.broadcasted_iota(jnp.int32, logits.shape, 1)
    m1 = jnp.max(logits, axis=1, keepdims=True)
    i1 = jnp.min(jnp.where(logits == m1, cols, E), axis=1, keepdims=True)
    masked = jnp.where(cols == i1, -jnp.inf, logits)
    m2 = jnp.max(masked, axis=1, keepdims=True)
    i2 = jnp.min(jnp.where(masked == m2, cols, E), axis=1, keepdims=True)
    e2 = jnp.exp(m2 - m1)
    denom = 1.0 + e2
    w1 = 1.0 / denom
    w2 = e2 / denom

    a = jnp.dot(x, a2_ref[...], preferred_element_type=jnp.float32)
    ecol = jax.lax.broadcasted_iota(jnp.int32, a.shape, 1) // R
    w_cols = jnp.where(ecol == i1, w1, 0.0) + jnp.where(ecol == i2, w2, 0.0)
    moe = jnp.dot(a * w_cols, b2_ref[...], preferred_element_type=jnp.float32)

    base = jnp.dot(x, wb_ref[...], preferred_element_type=jnp.float32)
    o_ref[...] = base + moe * SCALING


@jax.jit
def kernel(hidden_states, W_base, W_gate, lora_A, lora_B):
    A_all = lora_A.reshape(ER, D).T
    B_all = lora_B.transpose(0, 2, 1).reshape(ER, D)

    grid = (T // TILE_T,)
    return pl.pallas_call(
        _fused_kernel,
        grid=grid,
        in_specs=[
            pl.BlockSpec((TILE_T, D), lambda i: (i, 0)),
            pl.BlockSpec((D, D), lambda i: (0, 0)),
            pl.BlockSpec((D, E), lambda i: (0, 0)),
            pl.BlockSpec((D, ER), lambda i: (0, 0)),
            pl.BlockSpec((ER, D), lambda i: (0, 0)),
        ],
        out_specs=pl.BlockSpec((TILE_T, D), lambda i: (i, 0)),
        out_shape=jax.ShapeDtypeStruct((T, D), jnp.float32),
    )(hidden_states, W_base, W_gate, A_all, B_all)


# speedup vs baseline: 1.8253x; 1.0667x over previous
import jax
import jax.numpy as jnp
from jax.experimental import pallas as pl
from jax.experimental.pallas import tpu as pltpu

T = 2048
D = 2048
E = 8
R = 32
SCALING = 64 / 32
ER = E * R

TILE_T = 512
TILE_K = 512
NSLAB = D // TILE_K


def _fused_kernel(x_ref, wb_hbm, wg_ref, a2_ref, b2_ref, o_ref, wb_vmem, sem):
    i = pl.program_id(0)

    @pl.when(i == 0)
    def _start_stream():
        for k in range(NSLAB):
            pltpu.make_async_copy(
                wb_hbm.at[pl.ds(k * TILE_K, TILE_K), :],
                wb_vmem.at[pl.ds(k * TILE_K, TILE_K), :],
                sem.at[k],
            ).start()

    x = x_ref[...]
    logits = jnp.dot(x, wg_ref[...], preferred_element_type=jnp.float32)
    cols = jax.lax.broadcasted_iota(jnp.int32, logits.shape, 1)
    m1 = jnp.max(logits, axis=1, keepdims=True)
    i1 = jnp.min(jnp.where(logits == m1, cols, E), axis=1, keepdims=True)
    masked = jnp.where(cols == i1, -jnp.inf, logits)
    m2 = jnp.max(masked, axis=1, keepdims=True)
    i2 = jnp.min(jnp.where(masked == m2, cols, E), axis=1, keepdims=True)
    e2 = jnp.exp(m2 - m1)
    denom = 1.0 + e2
    w1 = 1.0 / denom
    w2 = e2 / denom

    a = jnp.dot(x, a2_ref[...], preferred_element_type=jnp.float32)
    ecol = jax.lax.broadcasted_iota(jnp.int32, a.shape, 1) // R
    w_cols = jnp.where(ecol == i1, w1, 0.0) + jnp.where(ecol == i2, w2, 0.0)
    moe = jnp.dot(a * w_cols, b2_ref[...], preferred_element_type=jnp.float32)

    @pl.when(i == 0)
    def _base_streamed():
        o_ref[...] = moe * SCALING
        for k in range(NSLAB):
            pltpu.make_async_copy(
                wb_hbm.at[pl.ds(k * TILE_K, TILE_K), :],
                wb_vmem.at[pl.ds(k * TILE_K, TILE_K), :],
                sem.at[k],
            ).wait()
            o_ref[...] += jnp.dot(
                x[:, k * TILE_K:(k + 1) * TILE_K],
                wb_vmem[pl.ds(k * TILE_K, TILE_K), :],
                preferred_element_type=jnp.float32)

    @pl.when(i > 0)
    def _base_resident():
        base = jnp.dot(x, wb_vmem[...], preferred_element_type=jnp.float32)
        o_ref[...] = base + moe * SCALING


@jax.jit
def kernel(hidden_states, W_base, W_gate, lora_A, lora_B):
    A_all = lora_A.reshape(ER, D).T
    B_all = lora_B.transpose(0, 2, 1).reshape(ER, D)

    grid = (T // TILE_T,)
    return pl.pallas_call(
        _fused_kernel,
        grid=grid,
        in_specs=[
            pl.BlockSpec((TILE_T, D), lambda i: (i, 0)),
            pl.BlockSpec(memory_space=pltpu.HBM),
            pl.BlockSpec((D, E), lambda i: (0, 0)),
            pl.BlockSpec((D, ER), lambda i: (0, 0)),
            pl.BlockSpec((ER, D), lambda i: (0, 0)),
        ],
        out_specs=pl.BlockSpec((TILE_T, D), lambda i: (i, 0)),
        out_shape=jax.ShapeDtypeStruct((T, D), jnp.float32),
        scratch_shapes=[
            pltpu.VMEM((D, D), jnp.float32),
            pltpu.SemaphoreType.DMA((NSLAB,)),
        ],
    )(hidden_states, W_base, W_gate, A_all, B_all)
